# baseline (speedup 1.0000x reference)
import jax
import jax.numpy as jnp
from jax import lax
from jax.experimental import pallas as pl
from jax.experimental.pallas import tpu as pltpu
from jax.experimental.pallas import tpu_sc as plsc

_B = 16384
_L = 200
_D = 32
_V = 1001
_W = _D // 2
_STRIDE = 17
_NC = 2
_NS = 16
_LANES = 16
_NW = _NC * _NS
_RPB = _LANES
_NBLK = _B // _RPB
_BPW = _NBLK // _NW
_RPP = 16
_LU = 4

_HI_MASK = -65536


def _sc_body(ids_hbm, tab_hbm, out_hbm, tab_v, ids_v, out_v, cnt_v):
    wid = lax.axis_index("s") * _NC + lax.axis_index("c")
    pltpu.sync_copy(tab_hbm, tab_v)
    word_iota = lax.iota(jnp.int32, _LANES)
    even_idx = word_iota * 2
    odd_idx = even_idx + 1
    lane_consts = [jnp.full((_LANES,), r, jnp.int32) for r in range(_LANES)]
    zf = jnp.zeros((_LANES,), jnp.float32)
    zi = jnp.zeros((_LANES,), jnp.int32)

    gdn = lax.GatherDimensionNumbers(
        offset_dims=(), collapsed_slice_dims=(0,), start_index_map=(0,))

    def splat(vec, r):
        return lax.gather(
            vec, lane_consts[r][:, None], gdn, (1,),
            mode=lax.GatherScatterMode.PROMISE_IN_BOUNDS)

    def block_body(i, carry):
        blk = wid * _BPW + i
        pltpu.sync_copy(ids_hbm.at[blk], ids_v)
        cnt_v[...] = zi
        inv = zf

        for p in range(1):
            r0 = p * _RPP
            init = tuple(zf for _ in range(2 * _RPP))

            def l_body(j, accs, p=p, r0=r0):
                pk = [None] * _RPP
                for u in range(_LU):
                    ids16 = ids_v[j * _LU + u]
                    if p == 0:
                        plsc.addupdate(
                            cnt_v.at[:], (ids16 != 0).astype(jnp.int32))
                    ids_s = ids16 * _STRIDE
                    for ri in range(_RPP):
                        s = splat(ids_s, r0 + ri)
                        g = plsc.load_gather(tab_v, [s + word_iota])
                        gbf = plsc.bitcast(g, jnp.bfloat16)
                        pk[ri] = gbf if u == 0 else pk[ri] + gbf
                out = []
                for ri in range(_RPP):
                    w = plsc.bitcast(pk[ri], jnp.int32)
                    lo = plsc.bitcast(w << 16, jnp.float32)
                    hi = plsc.bitcast(w & _HI_MASK, jnp.float32)
                    out.append(accs[2 * ri] + lo)
                    out.append(accs[2 * ri + 1] + hi)
                return tuple(out)

            accs = lax.fori_loop(0, _L // _LU, l_body, init)
            if p == 0:
                inv = 1.0 / jnp.maximum(cnt_v[...].astype(jnp.float32), 1.0)
            for ri in range(_RPP):
                r = r0 + ri
                inv_r = splat(inv, r)
                plsc.store_scatter(
                    out_v, [lane_consts[r], even_idx], accs[2 * ri] * inv_r)
                plsc.store_scatter(
                    out_v, [lane_consts[r], odd_idx], accs[2 * ri + 1] * inv_r)

        pltpu.sync_copy(out_v, out_hbm.at[pl.ds(blk * _RPB, _RPB)])
        return carry

    lax.fori_loop(0, _BPW, block_body, 0)


@jax.jit
def kernel(genre_ids_batch, embedding_weight):
    ids_t = genre_ids_batch.reshape(_NBLK, _RPB, _L).transpose(0, 2, 1)
    tab_bf = embedding_weight.astype(jnp.bfloat16).reshape(_V, _W, 2)
    tab_packed = jnp.pad(
        lax.bitcast_convert_type(tab_bf, jnp.int32),
        ((0, 0), (0, _STRIDE - _W))).reshape(_V * _STRIDE)
    call = pl.kernel(
        _sc_body,
        out_type=jax.ShapeDtypeStruct((_B, _D), jnp.float32),
        mesh=plsc.VectorSubcoreMesh(
            core_axis_name="c", subcore_axis_name="s",
            num_cores=_NC, num_subcores=_NS),
        scratch_types=[
            pltpu.VMEM((_V * _STRIDE,), jnp.int32),
            pltpu.VMEM((_L, _RPB), jnp.int32),
            pltpu.VMEM((_RPB, _D), jnp.float32),
            pltpu.VMEM((_LANES,), jnp.int32),
        ],
        compiler_params=pltpu.CompilerParams(
            use_tc_tiling_on_sc=False, needs_layout_passes=False),
    )
    return call(ids_t, tab_packed)


# speedup vs baseline: 1.2896x; 1.2896x over previous
import jax
import jax.numpy as jnp
from jax import lax
from jax.experimental import pallas as pl
from jax.experimental.pallas import tpu as pltpu
from jax.experimental.pallas import tpu_sc as plsc

_B = 16384
_L = 200
_D = 32
_V = 1001
_W = _D // 2
_STRIDE = 17
_NC = 2
_NS = 16
_LANES = 16
_NW = _NC * _NS
_RPB = _LANES
_NBLK = _B // _RPB
_BPW = _NBLK // _NW
_RPP = 8
_LU = 4

_HI_MASK = -65536


def _sc_body(ids_hbm, tab_hbm, out_hbm, tab_v, ids_v, out_v, cnt_v):
    wid = lax.axis_index("s") * _NC + lax.axis_index("c")
    pltpu.sync_copy(tab_hbm, tab_v)
    word_iota = lax.iota(jnp.int32, _LANES)
    row_iota = word_iota
    even_idx = word_iota * 2
    odd_idx = even_idx + 1
    lane_consts = [jnp.full((_LANES,), r, jnp.int32) for r in range(_LANES)]
    zf = jnp.zeros((_LANES,), jnp.float32)
    zi = jnp.zeros((_LANES,), jnp.int32)

    gdn = lax.GatherDimensionNumbers(
        offset_dims=(), collapsed_slice_dims=(0,), start_index_map=(0,))

    def splat(vec, r):
        return lax.gather(
            vec, lane_consts[r][:, None], gdn, (1,),
            mode=lax.GatherScatterMode.PROMISE_IN_BOUNDS)

    def block_body(i, carry):
        blk = wid * _BPW + i
        pltpu.sync_copy(ids_hbm.at[pl.ds(blk * _RPB, _RPB)], ids_v)
        cnt_v[...] = zi
        inv = zf

        for p in range(2):
            r0 = p * _RPP
            init = tuple(zf for _ in range(2 * _RPP))

            def l_body(j, accs, p=p, r0=r0):
                pk = [None] * _RPP
                for u in range(_LU):
                    lpos = jnp.full((_LANES,), j * _LU + u, jnp.int32)
                    ids16 = plsc.load_gather(ids_v, [row_iota, lpos])
                    if p == 0:
                        plsc.addupdate(
                            cnt_v.at[:], (ids16 != 0).astype(jnp.int32))
                    ids_s = ids16 * _STRIDE
                    for ri in range(_RPP):
                        s = splat(ids_s, r0 + ri)
                        g = plsc.load_gather(tab_v, [s + word_iota])
                        gbf = plsc.bitcast(g, jnp.bfloat16)
                        pk[ri] = gbf if u == 0 else pk[ri] + gbf
                out = []
                for ri in range(_RPP):
                    w = plsc.bitcast(pk[ri], jnp.int32)
                    lo = plsc.bitcast(w << 16, jnp.float32)
                    hi = plsc.bitcast(w & _HI_MASK, jnp.float32)
                    out.append(accs[2 * ri] + lo)
                    out.append(accs[2 * ri + 1] + hi)
                return tuple(out)

            accs = lax.fori_loop(0, _L // _LU, l_body, init)
            if p == 0:
                inv = 1.0 / jnp.maximum(cnt_v[...].astype(jnp.float32), 1.0)
            for ri in range(_RPP):
                r = r0 + ri
                inv_r = splat(inv, r)
                plsc.store_scatter(
                    out_v, [lane_consts[r], even_idx], accs[2 * ri] * inv_r)
                plsc.store_scatter(
                    out_v, [lane_consts[r], odd_idx], accs[2 * ri + 1] * inv_r)

        pltpu.sync_copy(out_v, out_hbm.at[pl.ds(blk * _RPB, _RPB)])
        return carry

    lax.fori_loop(0, _BPW, block_body, 0)


@jax.jit
def kernel(genre_ids_batch, embedding_weight):
    tab_bf = embedding_weight.astype(jnp.bfloat16).reshape(_V, _W, 2)
    tab_packed = jnp.pad(
        lax.bitcast_convert_type(tab_bf, jnp.int32),
        ((0, 0), (0, _STRIDE - _W))).reshape(_V * _STRIDE)
    call = pl.kernel(
        _sc_body,
        out_type=jax.ShapeDtypeStruct((_B, _D), jnp.float32),
        mesh=plsc.VectorSubcoreMesh(
            core_axis_name="c", subcore_axis_name="s",
            num_cores=_NC, num_subcores=_NS),
        scratch_types=[
            pltpu.VMEM((_V * _STRIDE,), jnp.int32),
            pltpu.VMEM((_RPB, _L), jnp.int32),
            pltpu.VMEM((_RPB, _D), jnp.float32),
            pltpu.VMEM((_LANES,), jnp.int32),
        ],
        compiler_params=pltpu.CompilerParams(
            use_tc_tiling_on_sc=False, needs_layout_passes=False),
    )
    return call(genre_ids_batch, tab_packed)


# speedup vs baseline: 1.4748x; 1.1437x over previous
import jax
import jax.numpy as jnp
from jax import lax
from jax.experimental import pallas as pl
from jax.experimental.pallas import tpu as pltpu
from jax.experimental.pallas import tpu_sc as plsc

_B = 16384
_L = 200
_D = 32
_V = 1001
_W = _D // 2
_STRIDE = 17
_NC = 2
_NS = 16
_LANES = 16
_NW = _NC * _NS
_RPB = _LANES
_NBLK = _B // _RPB
_BPW = _NBLK // _NW
_RPP = 8
_LU = 4

_HI_MASK = -65536


def _sc_body(ids_hbm, tab_hbm, out_hbm, tab_v, ids_v, out_v, cnt_v, sem):
    wid = lax.axis_index("s") * _NC + lax.axis_index("c")
    pltpu.async_copy(
        ids_hbm.at[pl.ds(wid * _BPW * _RPB, _RPB)], ids_v.at[0], sem)
    pltpu.sync_copy(tab_hbm, tab_v)
    word_iota = lax.iota(jnp.int32, _LANES)
    row_iota = word_iota
    even_idx = word_iota * 2
    odd_idx = even_idx + 1
    lane_consts = [jnp.full((_LANES,), r, jnp.int32) for r in range(_LANES)]
    zf = jnp.zeros((_LANES,), jnp.float32)
    zi = jnp.zeros((_LANES,), jnp.int32)

    gdn = lax.GatherDimensionNumbers(
        offset_dims=(), collapsed_slice_dims=(0,), start_index_map=(0,))

    def splat(vec, r):
        return lax.gather(
            vec, lane_consts[r][:, None], gdn, (1,),
            mode=lax.GatherScatterMode.PROMISE_IN_BOUNDS)

    def block_body(i, carry):
        blk = wid * _BPW + i
        buf = lax.rem(i, 2)
        pltpu.make_async_copy(
            ids_hbm.at[pl.ds(blk * _RPB, _RPB)], ids_v.at[buf], sem).wait()

        @pl.when(i + 1 < _BPW)
        def _prefetch():
            pltpu.async_copy(
                ids_hbm.at[pl.ds((blk + 1) * _RPB, _RPB)],
                ids_v.at[1 - buf], sem)

        bufv = jnp.full((_LANES,), buf, jnp.int32)
        cnt_v[...] = zi
        inv = zf

        for p in range(2):
            r0 = p * _RPP
            init = tuple(zf for _ in range(2 * _RPP))

            def l_body(j, accs, p=p, r0=r0):
                pk = [None] * _RPP
                for u in range(_LU):
                    lpos = jnp.full((_LANES,), j * _LU + u, jnp.int32)
                    ids16 = plsc.load_gather(ids_v, [bufv, row_iota, lpos])
                    if p == 0:
                        plsc.addupdate(
                            cnt_v.at[:], (ids16 != 0).astype(jnp.int32))
                    ids_s = ids16 * _STRIDE
                    for ri in range(_RPP):
                        s = splat(ids_s, r0 + ri)
                        g = plsc.load_gather(tab_v, [s + word_iota])
                        gbf = plsc.bitcast(g, jnp.bfloat16)
                        pk[ri] = gbf if u == 0 else pk[ri] + gbf
                out = []
                for ri in range(_RPP):
                    w = plsc.bitcast(pk[ri], jnp.int32)
                    lo = plsc.bitcast(w << 16, jnp.float32)
                    hi = plsc.bitcast(w & _HI_MASK, jnp.float32)
                    out.append(accs[2 * ri] + lo)
                    out.append(accs[2 * ri + 1] + hi)
                return tuple(out)

            accs = lax.fori_loop(0, _L // _LU, l_body, init)
            if p == 0:
                inv = 1.0 / jnp.maximum(cnt_v[...].astype(jnp.float32), 1.0)
            for ri in range(_RPP):
                r = r0 + ri
                inv_r = splat(inv, r)
                plsc.store_scatter(
                    out_v, [lane_consts[r], even_idx], accs[2 * ri] * inv_r)
                plsc.store_scatter(
                    out_v, [lane_consts[r], odd_idx], accs[2 * ri + 1] * inv_r)

        pltpu.sync_copy(out_v, out_hbm.at[pl.ds(blk * _RPB, _RPB)])
        return carry

    lax.fori_loop(0, _BPW, block_body, 0)


@jax.jit
def kernel(genre_ids_batch, embedding_weight):
    tab_bf = embedding_weight.astype(jnp.bfloat16).reshape(_V, _W, 2)
    tab_packed = jnp.pad(
        lax.bitcast_convert_type(tab_bf, jnp.int32),
        ((0, 0), (0, _STRIDE - _W))).reshape(_V * _STRIDE)
    call = pl.kernel(
        _sc_body,
        out_type=jax.ShapeDtypeStruct((_B, _D), jnp.float32),
        mesh=plsc.VectorSubcoreMesh(
            core_axis_name="c", subcore_axis_name="s",
            num_cores=_NC, num_subcores=_NS),
        scratch_types=[
            pltpu.VMEM((_V * _STRIDE,), jnp.int32),
            pltpu.VMEM((2, _RPB, _L), jnp.int32),
            pltpu.VMEM((_RPB, _D), jnp.float32),
            pltpu.VMEM((_LANES,), jnp.int32),
            pltpu.SemaphoreType.DMA,
        ],
        compiler_params=pltpu.CompilerParams(
            use_tc_tiling_on_sc=False, needs_layout_passes=False),
    )
    return call(genre_ids_batch, tab_packed)


# speedup vs baseline: 1.5019x; 1.0183x over previous
import jax
import jax.numpy as jnp
from jax import lax
from jax.experimental import pallas as pl
from jax.experimental.pallas import tpu as pltpu
from jax.experimental.pallas import tpu_sc as plsc

_B = 16384
_L = 200
_D = 32
_V = 1001
_W = _D // 2
_STRIDE = 17
_NC = 2
_NS = 16
_LANES = 16
_NW = _NC * _NS
_RPB = _LANES
_NBLK = _B // _RPB
_BPW = _NBLK // _NW
_RPP = 8
_LU = 4

_HI_MASK = -65536


def _sc_body(ids_hbm, tab_hbm, out_hbm, tab_v, ids_v, out_v, cnt_v, sem,
             sem_out):
    wid = lax.axis_index("s") * _NC + lax.axis_index("c")
    pltpu.async_copy(
        ids_hbm.at[pl.ds(wid * _BPW * _RPB, _RPB)], ids_v.at[0], sem)
    pltpu.sync_copy(tab_hbm, tab_v)
    word_iota = lax.iota(jnp.int32, _LANES)
    row_iota = word_iota
    even_idx = word_iota * 2
    odd_idx = even_idx + 1
    lane_consts = [jnp.full((_LANES,), r, jnp.int32) for r in range(_LANES)]
    zf = jnp.zeros((_LANES,), jnp.float32)
    zi = jnp.zeros((_LANES,), jnp.int32)

    gdn = lax.GatherDimensionNumbers(
        offset_dims=(), collapsed_slice_dims=(0,), start_index_map=(0,))

    def splat(vec, r):
        return lax.gather(
            vec, lane_consts[r][:, None], gdn, (1,),
            mode=lax.GatherScatterMode.PROMISE_IN_BOUNDS)

    def block_body(i, carry):
        blk = wid * _BPW + i
        buf = lax.rem(i, 2)
        pltpu.make_async_copy(
            ids_hbm.at[pl.ds(blk * _RPB, _RPB)], ids_v.at[buf], sem).wait()

        @pl.when(i + 1 < _BPW)
        def _prefetch():
            pltpu.async_copy(
                ids_hbm.at[pl.ds((blk + 1) * _RPB, _RPB)],
                ids_v.at[1 - buf], sem)

        bufv = jnp.full((_LANES,), buf, jnp.int32)

        @pl.when(i >= 2)
        def _drain_out():
            pltpu.make_async_copy(
                out_v.at[buf],
                out_hbm.at[pl.ds((blk - 2) * _RPB, _RPB)], sem_out).wait()

        cnt_v[...] = zi
        inv = zf

        for p in range(2):
            r0 = p * _RPP
            init = tuple(zf for _ in range(2 * _RPP))

            def l_body(j, accs, p=p, r0=r0):
                pk = [None] * _RPP
                for u in range(_LU):
                    lpos = jnp.full((_LANES,), j * _LU + u, jnp.int32)
                    ids16 = plsc.load_gather(ids_v, [bufv, row_iota, lpos])
                    if p == 0:
                        plsc.addupdate(
                            cnt_v.at[:], (ids16 != 0).astype(jnp.int32))
                    ids_s = ids16 * _STRIDE
                    for ri in range(_RPP):
                        s = splat(ids_s, r0 + ri)
                        g = plsc.load_gather(tab_v, [s + word_iota])
                        gbf = plsc.bitcast(g, jnp.bfloat16)
                        pk[ri] = gbf if u == 0 else pk[ri] + gbf
                out = []
                for ri in range(_RPP):
                    w = plsc.bitcast(pk[ri], jnp.int32)
                    lo = plsc.bitcast(w << 16, jnp.float32)
                    hi = plsc.bitcast(w & _HI_MASK, jnp.float32)
                    out.append(accs[2 * ri] + lo)
                    out.append(accs[2 * ri + 1] + hi)
                return tuple(out)

            accs = lax.fori_loop(0, _L // _LU, l_body, init)
            if p == 0:
                inv = 1.0 / jnp.maximum(cnt_v[...].astype(jnp.float32), 1.0)
            for ri in range(_RPP):
                r = r0 + ri
                inv_r = splat(inv, r)
                plsc.store_scatter(
                    out_v, [bufv, lane_consts[r], even_idx],
                    accs[2 * ri] * inv_r)
                plsc.store_scatter(
                    out_v, [bufv, lane_consts[r], odd_idx],
                    accs[2 * ri + 1] * inv_r)

        pltpu.async_copy(
            out_v.at[buf], out_hbm.at[pl.ds(blk * _RPB, _RPB)], sem_out)
        return carry

    lax.fori_loop(0, _BPW, block_body, 0)
    for k in (_BPW - 2, _BPW - 1):
        pltpu.make_async_copy(
            out_v.at[k % 2],
            out_hbm.at[pl.ds((wid * _BPW + k) * _RPB, _RPB)], sem_out).wait()


@jax.jit
def kernel(genre_ids_batch, embedding_weight):
    tab_bf = embedding_weight.astype(jnp.bfloat16).reshape(_V, _W, 2)
    tab_packed = jnp.pad(
        lax.bitcast_convert_type(tab_bf, jnp.int32),
        ((0, 0), (0, _STRIDE - _W))).reshape(_V * _STRIDE)
    call = pl.kernel(
        _sc_body,
        out_type=jax.ShapeDtypeStruct((_B, _D), jnp.float32),
        mesh=plsc.VectorSubcoreMesh(
            core_axis_name="c", subcore_axis_name="s",
            num_cores=_NC, num_subcores=_NS),
        scratch_types=[
            pltpu.VMEM((_V * _STRIDE,), jnp.int32),
            pltpu.VMEM((2, _RPB, _L), jnp.int32),
            pltpu.VMEM((2, _RPB, _D), jnp.float32),
            pltpu.VMEM((_LANES,), jnp.int32),
            pltpu.SemaphoreType.DMA,
            pltpu.SemaphoreType.DMA,
        ],
        compiler_params=pltpu.CompilerParams(
            use_tc_tiling_on_sc=False, needs_layout_passes=False),
    )
    return call(genre_ids_batch, tab_packed)
